# baseline (speedup 1.0000x reference)
import functools

import jax
import jax.numpy as jnp
from jax import lax
from jax.experimental import pallas as pl
from jax.experimental.pallas import tpu as pltpu
from jax.experimental.pallas import tpu_sc as plsc

_B, _L = 4096, 200
_N = _B * _L
_NC, _NS = 2, 16
_NW = _NC * _NS
_PER_W = _N // _NW
_CHUNK = _L
_NCHUNK = _PER_W // _CHUNK

_LABEL_VOCAB, _POS_VOCAB, _DEP_VOCAB = 42, 48, 42
_LABEL_DIM, _POS_DIM, _DEP_DIM = 64, 32, 32


def _build():
    mesh = plsc.VectorSubcoreMesh(core_axis_name="c", subcore_axis_name="s")

    @functools.partial(
        pl.kernel,
        out_type=jax.ShapeDtypeStruct((_B, _L, 128), jnp.float32),
        mesh=mesh,
        compiler_params=pltpu.CompilerParams(use_tc_tiling_on_sc=False,
                                             needs_layout_passes=False),
        scratch_types=(
            pltpu.VMEM_SHARED((_POS_VOCAB, _POS_DIM), jnp.float32),
            pltpu.VMEM_SHARED((_DEP_VOCAB, _DEP_DIM), jnp.float32),
            pltpu.VMEM_SHARED((_LABEL_VOCAB, _LABEL_DIM), jnp.float32),
            [pltpu.VMEM((_CHUNK,), jnp.int32) for _ in range(2)],
            [pltpu.VMEM((_CHUNK, _POS_DIM), jnp.float32) for _ in range(2)],
            [pltpu.VMEM((_CHUNK,), jnp.int32) for _ in range(2)],
            [pltpu.VMEM((_CHUNK, _DEP_DIM), jnp.float32) for _ in range(2)],
            [pltpu.VMEM((_CHUNK,), jnp.int32) for _ in range(2)],
            [pltpu.VMEM((_CHUNK, _LABEL_DIM), jnp.float32) for _ in range(2)],
            [pltpu.SemaphoreType.DMA for _ in range(3)],
            [pltpu.SemaphoreType.DMA for _ in range(3)],
            [pltpu.SemaphoreType.DMA for _ in range(3)],
            pltpu.SemaphoreType.DMA,
        ),
    )
    def emb_kernel(label_ids, pos_ids, dep_ids, label_tab, pos_tab, dep_tab,
                   comb_out,
                   pos_tv, dep_tv, lab_tv,
                   pos_idx, pos_rows, dep_idx, dep_rows, lab_idx, lab_rows,
                   sl, sg, ss, st):
        wid = lax.axis_index("s") * _NC + lax.axis_index("c")
        w_base = wid * _PER_W
        w_row = wid * _NCHUNK

        @pl.when(lax.axis_index("s") == 0)
        def _():
            pltpu.make_async_copy(pos_tab, pos_tv, st).start()
            pltpu.make_async_copy(dep_tab, dep_tv, st).start()
            pltpu.make_async_copy(label_tab, lab_tv, st).start()
            pltpu.make_async_copy(pos_tab, pos_tv, st).wait()
            pltpu.make_async_copy(dep_tab, dep_tv, st).wait()
            pltpu.make_async_copy(label_tab, lab_tv, st).wait()

        plsc.subcore_barrier()

        tables = (
            (pos_ids, pos_tv, 0, _POS_DIM, pos_idx, pos_rows, 0),
            (dep_ids, dep_tv, _POS_DIM, _DEP_DIM, dep_idx, dep_rows, 1),
            (label_ids, lab_tv, _POS_DIM + _DEP_DIM, _LABEL_DIM, lab_idx, lab_rows, 2),
        )

        def out_band(i, off, dim):
            return comb_out.at[w_row + i, :, pl.ds(off, dim)]

        def chunk_slice(i):
            return pl.ds(w_base + i * _CHUNK, _CHUNK)

        def step(i, b, first, second):
            nb = 1 - b
            for ids, tab, off, dim, idx, rows, t in tables:
                if not (first or second):
                    pltpu.make_async_copy(rows[b], out_band(i, off, dim), ss[t]).wait()
                if not first:
                    pltpu.make_async_copy(tab.at[idx[nb]], rows[nb], sg[t]).wait()
                    pltpu.make_async_copy(rows[nb], out_band(i - 1, off, dim),
                                          ss[t]).start()
                pltpu.make_async_copy(ids.at[chunk_slice(i)], idx[b], sl[t]).wait()
                pltpu.make_async_copy(tab.at[idx[b]], rows[b], sg[t]).start()
                if isinstance(i, int):
                    if i < _NCHUNK - 1:
                        pltpu.make_async_copy(ids.at[chunk_slice(i + 1)], idx[nb],
                                              sl[t]).start()
                else:
                    @pl.when(i < _NCHUNK - 1)
                    def _():
                        pltpu.make_async_copy(ids.at[chunk_slice(i + 1)], idx[nb],
                                              sl[t]).start()

        for ids, tab, off, dim, idx, rows, t in tables:
            pltpu.make_async_copy(ids.at[chunk_slice(0)], idx[0], sl[t]).start()

        step(0, 0, True, False)
        step(1, 1, False, True)

        def body(i2, carry):
            step(2 * i2, 0, False, False)
            step(2 * i2 + 1, 1, False, False)
            return carry

        lax.fori_loop(1, _NCHUNK // 2, body, 0)

        last = _NCHUNK - 1
        for ids, tab, off, dim, idx, rows, t in tables:
            pltpu.make_async_copy(tab.at[idx[1]], rows[1], sg[t]).wait()
            pltpu.make_async_copy(rows[1], out_band(last, off, dim), ss[t]).start()
            pltpu.make_async_copy(rows[0], out_band(last - 1, off, dim), ss[t]).wait()
            pltpu.make_async_copy(rows[1], out_band(last, off, dim), ss[t]).wait()

    return emb_kernel


_EMB = _build()


def kernel(label_ids, pos_ids, dep_ids, label_table, pos_table, dep_table):
    lab = label_ids.reshape(_N).astype(jnp.int32)
    pos = pos_ids.reshape(_N).astype(jnp.int32)
    dep = dep_ids.reshape(_N).astype(jnp.int32)
    comb = _EMB(lab, pos, dep, label_table, pos_table, dep_table)
    return (
        comb[:, :, :_POS_DIM],
        comb[:, :, _POS_DIM:_POS_DIM + _DEP_DIM],
        comb[:, :, _POS_DIM + _DEP_DIM:],
    )


# speedup vs baseline: 1.2759x; 1.2759x over previous
import functools

import jax
import jax.numpy as jnp
from jax import lax
from jax.experimental import pallas as pl
from jax.experimental.pallas import tpu as pltpu
from jax.experimental.pallas import tpu_sc as plsc

_B, _L = 4096, 200
_N = _B * _L
_NC, _NS = 2, 16
_NW = _NC * _NS
_PER_W = _N // _NW
_CHUNK = _L
_NCHUNK = _PER_W // _CHUNK

_LABEL_VOCAB, _POS_VOCAB, _DEP_VOCAB = 42, 48, 42
_LABEL_DIM, _POS_DIM, _DEP_DIM = 64, 32, 32


def _build():
    mesh = plsc.VectorSubcoreMesh(core_axis_name="c", subcore_axis_name="s")

    @functools.partial(
        pl.kernel,
        out_type=(
            jax.ShapeDtypeStruct((_B, _L, _POS_DIM), jnp.float32),
            jax.ShapeDtypeStruct((_B, _L, _DEP_DIM), jnp.float32),
            jax.ShapeDtypeStruct((_B, _L, _LABEL_DIM), jnp.float32),
        ),
        mesh=mesh,
        compiler_params=pltpu.CompilerParams(use_tc_tiling_on_sc=False,
                                             needs_layout_passes=False),
        scratch_types=(
            pltpu.VMEM_SHARED((_POS_VOCAB, _POS_DIM), jnp.float32),
            pltpu.VMEM_SHARED((_DEP_VOCAB, _DEP_DIM), jnp.float32),
            pltpu.VMEM_SHARED((_LABEL_VOCAB, _LABEL_DIM), jnp.float32),
            [pltpu.VMEM((_CHUNK,), jnp.int32) for _ in range(2)],
            [pltpu.VMEM((_CHUNK, _POS_DIM), jnp.float32) for _ in range(2)],
            [pltpu.VMEM((_CHUNK,), jnp.int32) for _ in range(2)],
            [pltpu.VMEM((_CHUNK, _DEP_DIM), jnp.float32) for _ in range(2)],
            [pltpu.VMEM((_CHUNK,), jnp.int32) for _ in range(2)],
            [pltpu.VMEM((_CHUNK, _LABEL_DIM), jnp.float32) for _ in range(2)],
            [pltpu.SemaphoreType.DMA for _ in range(3)],
            [pltpu.SemaphoreType.DMA for _ in range(3)],
            [pltpu.SemaphoreType.DMA for _ in range(3)],
            pltpu.SemaphoreType.DMA,
        ),
    )
    def emb_kernel(label_ids, pos_ids, dep_ids, label_tab, pos_tab, dep_tab,
                   pos_out, dep_out, label_out,
                   pos_tv, dep_tv, lab_tv,
                   pos_idx, pos_rows, dep_idx, dep_rows, lab_idx, lab_rows,
                   sl, sg, ss, st):
        wid = lax.axis_index("s") * _NC + lax.axis_index("c")
        w_base = wid * _PER_W
        w_row = wid * _NCHUNK

        @pl.when(lax.axis_index("s") == 0)
        def _():
            pltpu.make_async_copy(pos_tab, pos_tv, st).start()
            pltpu.make_async_copy(dep_tab, dep_tv, st).start()
            pltpu.make_async_copy(label_tab, lab_tv, st).start()
            pltpu.make_async_copy(pos_tab, pos_tv, st).wait()
            pltpu.make_async_copy(dep_tab, dep_tv, st).wait()
            pltpu.make_async_copy(label_tab, lab_tv, st).wait()

        plsc.subcore_barrier()

        tables = (
            (pos_ids, pos_tv, pos_out, pos_idx, pos_rows, 0),
            (dep_ids, dep_tv, dep_out, dep_idx, dep_rows, 1),
            (label_ids, lab_tv, label_out, lab_idx, lab_rows, 2),
        )

        def chunk_slice(i):
            return pl.ds(w_base + i * _CHUNK, _CHUNK)

        def step(i, b, first, second):
            nb = 1 - b
            for ids, tab, out, idx, rows, t in tables:
                if not (first or second):
                    pltpu.make_async_copy(rows[b], out.at[w_row + i], ss[t]).wait()
                if not first:
                    pltpu.make_async_copy(tab.at[idx[nb]], rows[nb], sg[t]).wait()
                    pltpu.make_async_copy(rows[nb], out.at[w_row + i - 1],
                                          ss[t]).start()
                pltpu.make_async_copy(ids.at[chunk_slice(i)], idx[b], sl[t]).wait()
                pltpu.make_async_copy(tab.at[idx[b]], rows[b], sg[t]).start()
                if isinstance(i, int):
                    if i < _NCHUNK - 1:
                        pltpu.make_async_copy(ids.at[chunk_slice(i + 1)], idx[nb],
                                              sl[t]).start()
                else:
                    @pl.when(i < _NCHUNK - 1)
                    def _():
                        pltpu.make_async_copy(ids.at[chunk_slice(i + 1)], idx[nb],
                                              sl[t]).start()

        for ids, tab, out, idx, rows, t in tables:
            pltpu.make_async_copy(ids.at[chunk_slice(0)], idx[0], sl[t]).start()

        step(0, 0, True, False)
        step(1, 1, False, True)

        def body(i2, carry):
            step(2 * i2, 0, False, False)
            step(2 * i2 + 1, 1, False, False)
            return carry

        lax.fori_loop(1, _NCHUNK // 2, body, 0)

        last = _NCHUNK - 1
        for ids, tab, out, idx, rows, t in tables:
            pltpu.make_async_copy(tab.at[idx[1]], rows[1], sg[t]).wait()
            pltpu.make_async_copy(rows[1], out.at[w_row + last], ss[t]).start()
            pltpu.make_async_copy(rows[0], out.at[w_row + last - 1], ss[t]).wait()
            pltpu.make_async_copy(rows[1], out.at[w_row + last], ss[t]).wait()

    return emb_kernel


_EMB = _build()


def kernel(label_ids, pos_ids, dep_ids, label_table, pos_table, dep_table):
    lab = label_ids.reshape(_N).astype(jnp.int32)
    pos = pos_ids.reshape(_N).astype(jnp.int32)
    dep = dep_ids.reshape(_N).astype(jnp.int32)
    return _EMB(lab, pos, dep, label_table, pos_table, dep_table)
